# baseline (speedup 1.0000x reference)
import functools

import jax
import jax.numpy as jnp
from jax import lax
from jax.experimental import pallas as pl
from jax.experimental.pallas import tpu as pltpu
from jax.experimental.pallas import tpu_sc as plsc

_BSZ, _SEQ, _D = 8, 2048, 1024
_WSEG = 32
_NW = _SEQ // _WSEG
_NWT = _NW // 16
_AW = _WSEG * _D


def _sc_body(meta_hbm, pack_hbm, hid_hbm, zer_hbm, out_hbm,
             acc, inbuf, sidx_v, meta_v):
    c = lax.axis_index("c")
    s = lax.axis_index("s")

    qiota = lax.iota(jnp.int32, 16)

    def _extract(vec, j):
        return jnp.max(jnp.where(qiota == j, vec, jnp.int32(-2**31)))

    def _pair(m, carry):
        b = c * 4 + m // 4
        w = s + (m % 4) * 16

        pltpu.sync_copy(zer_hbm, acc)

        pltpu.sync_copy(meta_hbm.at[pl.ds((b * _NW + w) * 16, 16)], meta_v)
        t0 = _extract(meta_v[...], 0)
        nch = _extract(meta_v[...], 1)

        def _chunk(k, carry2):
            cb = pl.multiple_of((b * _SEQ + t0 + k * 16) * _D, 16 * _D)
            pltpu.sync_copy(hid_hbm.at[pl.ds(cb, 16 * _D)], inbuf)
            cp = pl.multiple_of(b * _SEQ + t0 + k * 16, 16)
            pltpu.sync_copy(pack_hbm.at[pl.ds(cp, 16)], sidx_v)
            sv = sidx_v[...]
            for j in range(16):
                sb = _extract(sv, j)

                @pl.when((sb >> 16) == w)
                def _():
                    base = (sb & 0xFFFF) * _D

                    def _q4(q, carry3):
                        for u in range(4):
                            off = q * 64 + u * 16
                            idx = jnp.full((16,), base + off,
                                           jnp.int32) + qiota
                            plsc.addupdate_scatter(
                                acc, [idx],
                                inbuf[pl.ds(j * _D + off, 16)])
                        return carry3
                    lax.fori_loop(0, 16, _q4, 0)
            return carry2
        lax.fori_loop(0, nch, _chunk, 0)

        o0 = pl.multiple_of((b * _SEQ + w * _WSEG) * _D, _AW)
        pltpu.sync_copy(acc, out_hbm.at[pl.ds(o0, _AW)])
        return carry
    lax.fori_loop(0, 4 * _NWT, _pair, 0)


def kernel(domains, hidden_layers, binary_preditions):
    del domains
    labels = binary_preditions
    is_B = (labels == 1).astype(jnp.int32)
    is_I = labels == 2
    cs = jnp.cumsum(is_B, axis=1)
    seg = cs - 1
    valid = ((is_B == 1) | is_I) & (seg >= 0)

    pack = jnp.where(valid, (seg // _WSEG) * 65536 + seg % _WSEG, -1)
    pack = pack.astype(jnp.int32).reshape(-1)

    bounds = jnp.arange(_NW + 1, dtype=jnp.int32) * _WSEG
    below = (seg[:, None, :] < bounds[None, :, None]).sum(-1)
    t_lo = (below[:, :-1] // 16) * 16
    nch = (below[:, 1:] - t_lo + 15) // 16
    meta = jnp.stack(
        [t_lo, nch] + [jnp.zeros_like(t_lo)] * 14, axis=-1)
    meta = meta.astype(jnp.int32).reshape(-1)

    hid_flat = hidden_layers.reshape(-1)
    zer = jnp.zeros((_AW,), jnp.float32)

    mesh = plsc.VectorSubcoreMesh(core_axis_name="c", subcore_axis_name="s")
    sc = functools.partial(
        pl.kernel,
        mesh=mesh,
        compiler_params=pltpu.CompilerParams(needs_layout_passes=False),
        out_type=jax.ShapeDtypeStruct((_BSZ * _SEQ * _D,), jnp.float32),
        scratch_types=[
            pltpu.VMEM((_AW,), jnp.float32),
            pltpu.VMEM((16 * _D,), jnp.float32),
            pltpu.VMEM((16,), jnp.int32),
            pltpu.VMEM((16,), jnp.int32),
        ],
    )(_sc_body)
    return sc(meta, pack, hid_flat, zer).reshape(_BSZ * _SEQ, _D)


# speedup vs baseline: 1.0741x; 1.0741x over previous
import functools

import jax
import jax.numpy as jnp
from jax import lax
from jax.experimental import pallas as pl
from jax.experimental.pallas import tpu as pltpu
from jax.experimental.pallas import tpu_sc as plsc

_BSZ, _SEQ, _D = 8, 2048, 1024
_WSEG = 16
_NW = _SEQ // _WSEG
_NWT = _NW // 16
_AW = _WSEG * _D


def _sc_body(meta_hbm, pack_hbm, hid_hbm, zer_hbm, out_hbm,
             acc, inbuf, zbuf, sidx_v, meta_v):
    c = lax.axis_index("c")
    s = lax.axis_index("s")

    pltpu.sync_copy(zer_hbm, zbuf)

    qiota = lax.iota(jnp.int32, 16)

    def _extract(vec, j):
        return jnp.max(jnp.where(qiota == j, vec, jnp.int32(-2**31)))

    def _pair(m, carry):
        b = c * 4 + m // _NWT
        w = s + (m % _NWT) * 16

        pltpu.sync_copy(meta_hbm.at[pl.ds((b * _NW + w) * 16, 16)], meta_v)
        t0 = _extract(meta_v[...], 0)
        nch = _extract(meta_v[...], 1)

        o0 = pl.multiple_of((b * _SEQ + w * _WSEG) * _D, _AW)

        @pl.when(nch == 0)
        def _():
            pltpu.sync_copy(zbuf, out_hbm.at[pl.ds(o0, _AW)])

        @pl.when(nch > 0)
        def _():
            pltpu.sync_copy(zer_hbm, acc)

            def _chunk(k, carry2):
                cb = pl.multiple_of((b * _SEQ + t0 + k * 16) * _D, 16 * _D)
                pltpu.sync_copy(hid_hbm.at[pl.ds(cb, 16 * _D)], inbuf)
                cp = pl.multiple_of(b * _SEQ + t0 + k * 16, 16)
                pltpu.sync_copy(pack_hbm.at[pl.ds(cp, 16)], sidx_v)
                sv = sidx_v[...]
                for j in range(16):
                    sb = _extract(sv, j)

                    @pl.when((sb >> 16) == w)
                    def _():
                        base = jnp.full((16,), (sb & 0xFFFF) * _D,
                                        jnp.int32) + qiota

                        def _q2(q, carry3):
                            for u in range(32):
                                off = q * 512 + u * 16
                                plsc.addupdate_scatter(
                                    acc, [base + off],
                                    inbuf[pl.ds(j * _D + off, 16)])
                            return carry3
                        lax.fori_loop(0, 2, _q2, 0)
                return carry2
            lax.fori_loop(0, nch, _chunk, 0)

            pltpu.sync_copy(acc, out_hbm.at[pl.ds(o0, _AW)])
        return carry
    lax.fori_loop(0, 4 * _NWT, _pair, 0)


def kernel(domains, hidden_layers, binary_preditions):
    del domains
    labels = binary_preditions
    is_B = (labels == 1).astype(jnp.int32)
    is_I = labels == 2
    cs = jnp.cumsum(is_B, axis=1)
    seg = cs - 1
    valid = ((is_B == 1) | is_I) & (seg >= 0)

    pack = jnp.where(valid, (seg // _WSEG) * 65536 + seg % _WSEG, -1)
    pack = pack.astype(jnp.int32).reshape(-1)

    bounds = jnp.arange(_NW + 1, dtype=jnp.int32) * _WSEG
    below = (seg[:, None, :] < bounds[None, :, None]).sum(-1)
    t_lo = (below[:, :-1] // 16) * 16
    nch = (below[:, 1:] - t_lo + 15) // 16
    meta = jnp.stack(
        [t_lo, nch] + [jnp.zeros_like(t_lo)] * 14, axis=-1)
    meta = meta.astype(jnp.int32).reshape(-1)

    hid_flat = hidden_layers.reshape(-1)
    zer = jnp.zeros((_AW,), jnp.float32)

    mesh = plsc.VectorSubcoreMesh(core_axis_name="c", subcore_axis_name="s")
    sc = functools.partial(
        pl.kernel,
        mesh=mesh,
        compiler_params=pltpu.CompilerParams(needs_layout_passes=False),
        out_type=jax.ShapeDtypeStruct((_BSZ * _SEQ * _D,), jnp.float32),
        scratch_types=[
            pltpu.VMEM((_AW,), jnp.float32),
            pltpu.VMEM((16 * _D,), jnp.float32),
            pltpu.VMEM((_AW,), jnp.float32),
            pltpu.VMEM((16,), jnp.int32),
            pltpu.VMEM((16,), jnp.int32),
        ],
    )(_sc_body)
    return sc(meta, pack, hid_flat, zer).reshape(_BSZ * _SEQ, _D)
